# baseline (speedup 1.0000x reference)
import functools

import jax
import jax.numpy as jnp
from jax import lax
from jax.experimental import pallas as pl
from jax.experimental.pallas import tpu as pltpu
from jax.experimental.pallas import tpu_sc as plsc

_EMBED_DIM = 64
_NUM_WORKERS = 32
_SUB = 128
_SUP = 512
_N_SUB = _SUP // _SUB


def _make_emb_kernel(b_total: int):
    b_per_w = b_total // _NUM_WORKERS
    n_sup = b_per_w // _SUP
    mesh = plsc.VectorSubcoreMesh(core_axis_name="c", subcore_axis_name="s")

    @functools.partial(
        pl.kernel,
        mesh=mesh,
        out_type=jax.ShapeDtypeStruct((b_total, _EMBED_DIM), jnp.float32),
        scratch_types=[
            pltpu.VMEM((b_per_w,), jnp.int32),
            pltpu.VMEM((_SUP, _EMBED_DIM), jnp.float32),
            pltpu.SemaphoreType.DMA,
        ],
        compiler_params=pltpu.CompilerParams(use_tc_tiling_on_sc=False),
    )
    def emb(weight_hbm, x_hbm, out_hbm, idx_v, buf, gsem):
        wid = lax.axis_index("s") * 2 + lax.axis_index("c")
        base = wid * b_per_w
        pltpu.sync_copy(x_hbm.at[pl.ds(base, b_per_w)], idx_v)

        def body(g, carry):
            off = g * _SUP
            copies = [
                pltpu.async_copy(
                    weight_hbm.at[idx_v.at[pl.ds(off + j * _SUB, _SUB)]],
                    buf.at[pl.ds(j * _SUB, _SUB)],
                    gsem,
                )
                for j in range(_N_SUB)
            ]
            for c in copies:
                c.wait()
            pltpu.sync_copy(buf, out_hbm.at[pl.ds(base + off, _SUP)])
            return carry

        lax.fori_loop(0, n_sup, body, 0)

    return emb


@jax.jit
def kernel(x, weight):
    b, h = x.shape
    xf = x.reshape(-1).astype(jnp.int32)
    out = _make_emb_kernel(b * h)(weight, xf)
    return out.reshape(b, h, _EMBED_DIM)


# speedup vs baseline: 1.0214x; 1.0214x over previous
import functools

import jax
import jax.numpy as jnp
from jax import lax
from jax.experimental import pallas as pl
from jax.experimental.pallas import tpu as pltpu
from jax.experimental.pallas import tpu_sc as plsc

_EMBED_DIM = 64
_NUM_WORKERS = 32
_SUB = 128
_SUP = 256
_N_SUB = _SUP // _SUB
_NBUF = 4


def _make_emb_kernel(b_total: int):
    b_per_w = b_total // _NUM_WORKERS
    n_sup = b_per_w // _SUP
    n_iter = n_sup // _NBUF
    mesh = plsc.VectorSubcoreMesh(core_axis_name="c", subcore_axis_name="s")

    @functools.partial(
        pl.kernel,
        mesh=mesh,
        out_type=jax.ShapeDtypeStruct((b_total, _EMBED_DIM), jnp.float32),
        scratch_types=[
            pltpu.VMEM((b_per_w,), jnp.int32),
            *[pltpu.VMEM((_SUP, _EMBED_DIM), jnp.float32) for _ in range(_NBUF)],
            *[pltpu.SemaphoreType.DMA for _ in range(2 * _NBUF)],
        ],
        compiler_params=pltpu.CompilerParams(use_tc_tiling_on_sc=False),
    )
    def emb(weight_hbm, x_hbm, out_hbm, idx_v, *scratch):
        bufs = scratch[:_NBUF]
        gsems = scratch[_NBUF:2 * _NBUF]
        osems = scratch[2 * _NBUF:]
        wid = lax.axis_index("s") * 2 + lax.axis_index("c")
        base = wid * b_per_w
        pltpu.sync_copy(x_hbm.at[pl.ds(base, b_per_w)], idx_v)

        def issue_gather(g, s):
            off = g * _SUP
            for j in range(_N_SUB):
                pltpu.async_copy(
                    weight_hbm.at[idx_v.at[pl.ds(off + j * _SUB, _SUB)]],
                    bufs[s].at[pl.ds(j * _SUB, _SUB)],
                    gsems[s],
                )

        def wait_gather(s):
            pltpu.make_async_copy(
                weight_hbm.at[pl.ds(0, _SUP)], bufs[s], gsems[s]
            ).wait()

        def issue_write(g, s):
            pltpu.async_copy(
                bufs[s], out_hbm.at[pl.ds(base + g * _SUP, _SUP)], osems[s]
            )

        def drain_write(s):
            pltpu.make_async_copy(
                bufs[s], out_hbm.at[pl.ds(base, _SUP)], osems[s]
            ).wait()

        for s in range(_NBUF):
            issue_gather(s, s)

        def body(p, carry):
            for s in range(_NBUF):
                wait_gather(s)
                issue_write(p * _NBUF + s, s)
            for s in range(_NBUF):
                drain_write(s)
                issue_gather((p + 1) * _NBUF + s, s)
            return carry

        lax.fori_loop(0, n_iter - 1, body, 0)

        for s in range(_NBUF):
            wait_gather(s)
            issue_write((n_iter - 1) * _NBUF + s, s)
        for s in range(_NBUF):
            drain_write(s)

    return emb


@jax.jit
def kernel(x, weight):
    b, h = x.shape
    xf = x.reshape(-1).astype(jnp.int32)
    out = _make_emb_kernel(b * h)(weight, xf)
    return out.reshape(b, h, _EMBED_DIM)
